# baseline (speedup 1.0000x reference)
import jax
import jax.numpy as jnp
import numpy as np
from jax.experimental import pallas as pl
from jax.experimental.pallas import tpu as pltpu

_IDX = np.array(
    [3, 17, 29, 42, 56, 61, 73, 88, 91, 104, 111, 123, 130, 142, 150,
     158, 163, 171, 180, 187, 195, 7, 12, 25, 33, 47, 52, 66, 79, 83,
     96, 101, 115, 127, 135, 146, 153, 167, 174, 182, 190, 199, 5, 19,
     38, 59, 70, 99, 119, 139],
    dtype=np.int32,
)
_N_IDX = _IDX.shape[0]


def _copy_body(idx_ref, x_ref, o_ref):
    o_ref[...] = x_ref[...]


def kernel(x):
    B, R, F = x.shape
    x4 = x.reshape(B, R, 1, F)
    idx = jnp.asarray(_IDX)

    b_blk = 1024
    grid = (B // b_blk, _N_IDX)

    out = pl.pallas_call(
        _copy_body,
        grid_spec=pltpu.PrefetchScalarGridSpec(
            num_scalar_prefetch=1,
            grid=grid,
            in_specs=[
                pl.BlockSpec(
                    (b_blk, 1, 1, F), lambda b, j, idx_ref: (b, idx_ref[j], 0, 0)
                ),
            ],
            out_specs=pl.BlockSpec(
                (b_blk, 1, 1, F), lambda b, j, idx_ref: (b, j, 0, 0)
            ),
        ),
        out_shape=jax.ShapeDtypeStruct((B, _N_IDX, 1, F), x.dtype),
    )(idx, x4)
    return out.reshape(B, _N_IDX, F)


# speedup vs baseline: 1.1971x; 1.1971x over previous
import jax
import jax.numpy as jnp
import numpy as np
from jax.experimental import pallas as pl
from jax.experimental.pallas import tpu as pltpu

_IDX = np.array(
    [3, 17, 29, 42, 56, 61, 73, 88, 91, 104, 111, 123, 130, 142, 150,
     158, 163, 171, 180, 187, 195, 7, 12, 25, 33, 47, 52, 66, 79, 83,
     96, 101, 115, 127, 135, 146, 153, 167, 174, 182, 190, 199, 5, 19,
     38, 59, 70, 99, 119, 139],
    dtype=np.int32,
)
_N_IDX = _IDX.shape[0]


def _copy_body(idx_ref, x_ref, o_ref):
    o_ref[...] = x_ref[...]


def kernel(x):
    B, R, F = x.shape
    x4 = x.reshape(B, R, 1, F)
    idx = jnp.asarray(_IDX)

    b_blk = 8192
    grid = (B // b_blk, _N_IDX)

    out = pl.pallas_call(
        _copy_body,
        grid_spec=pltpu.PrefetchScalarGridSpec(
            num_scalar_prefetch=1,
            grid=grid,
            in_specs=[
                pl.BlockSpec(
                    (b_blk, 1, 1, F), lambda b, j, idx_ref: (b, idx_ref[j], 0, 0)
                ),
            ],
            out_specs=pl.BlockSpec(
                (b_blk, 1, 1, F), lambda b, j, idx_ref: (b, j, 0, 0)
            ),
        ),
        out_shape=jax.ShapeDtypeStruct((B, _N_IDX, 1, F), x.dtype),
    )(idx, x4)
    return out.reshape(B, _N_IDX, F)
